# Initial kernel scaffold; baseline (speedup 1.0000x reference)
import jax
import jax.numpy as jnp
from jax.experimental import pallas as pl


def kernel(x, edge_index, edge_attr, nn1_w1, nn1_b1, nn1_g, nn1_be, nn1_w2, nn1_b2, conv1_root, conv1_bias, nn2_w1, nn2_b1, nn2_g, nn2_be, nn2_w2, nn2_b2, conv2_root, conv2_bias, bn_g, bn_be):
    raise NotImplementedError("write your pallas kernel here")


# speedup vs baseline: 3.3582x; 3.3582x over previous
import functools

import jax
import jax.numpy as jnp
from jax import lax
from jax.experimental import pallas as pl
from jax.experimental.pallas import tpu as pltpu
from jax.experimental.pallas import tpu_sc as plsc

_NC = 2
_NS = 16
_NW = _NC * _NS
_CH = 128

_F32 = jnp.float32


def _gram_body(ea_ref, g_ref, s_ref):
    i = pl.program_id(0)

    @pl.when(i == 0)
    def _():
        g_ref[...] = jnp.zeros_like(g_ref)
        s_ref[...] = jnp.zeros_like(s_ref)

    ea = ea_ref[...]
    g_ref[...] += lax.dot_general(ea, ea, (((0,), (0,)), ((), ())),
                                  preferred_element_type=_F32)
    s_ref[...] += jnp.sum(ea, axis=0, keepdims=True)


def _gram(ea, tile):
    e, t = ea.shape
    return pl.pallas_call(
        _gram_body,
        grid=(e // tile,),
        in_specs=[pl.BlockSpec((tile, t), lambda i: (i, 0))],
        out_specs=[pl.BlockSpec((t, t), lambda i: (0, 0)),
                   pl.BlockSpec((1, t), lambda i: (0, 0))],
        out_shape=[jax.ShapeDtypeStruct((t, t), _F32),
                   jax.ShapeDtypeStruct((1, t), _F32)],
    )(ea)


def _edge_body(ea_ref, xj_ref, w1_ref, c1_ref, w2r_ref, exp_ref, sum_ref,
               b2m_ref, out_ref):
    ea = ea_ref[...]
    xj = xj_ref[...]
    h = jnp.maximum(
        jnp.dot(ea, w1_ref[...], preferred_element_type=_F32) + c1_ref[...],
        0.0)
    m2 = jnp.dot(xj, w2r_ref[...], preferred_element_type=_F32)
    hexp = jnp.dot(h, exp_ref[...], preferred_element_type=_F32)
    msg = jnp.dot(hexp * m2, sum_ref[...], preferred_element_type=_F32)
    out_ref[...] = msg + jnp.dot(xj, b2m_ref[...], preferred_element_type=_F32)


def _edge_messages(ea, xj, w1f, c1, w2r, expand, summ, b2m, tile):
    e, t = ea.shape
    f = xj.shape[1]
    dp = w2r.shape[1]
    d = summ.shape[1]
    small = lambda shape: pl.BlockSpec(shape, lambda i: (0, 0))
    return pl.pallas_call(
        _edge_body,
        grid=(e // tile,),
        in_specs=[pl.BlockSpec((tile, t), lambda i: (i, 0)),
                  pl.BlockSpec((tile, f), lambda i: (i, 0)),
                  small((t, d)), small((1, d)), small((f, dp)),
                  small((d, dp)), small((dp, d)), small((f, d))],
        out_specs=pl.BlockSpec((tile, d), lambda i: (i, 0)),
        out_shape=jax.ShapeDtypeStruct((e, d), _F32),
    )(ea, xj, w1f, c1, w2r, expand, summ, b2m)


def _combine_body(s0_ref, s1_ref, c0_ref, c1_ref, x_ref, root_ref, bias_ref,
                  g_ref, be_ref, out_ref, *, bn_relu):
    s = s0_ref[...] + s1_ref[...]
    cnt = (c0_ref[...] + c1_ref[...])[:, 0:1]
    agg = s / jnp.maximum(cnt, 1.0)
    out = agg + jnp.dot(x_ref[...], root_ref[...],
                        preferred_element_type=_F32) + bias_ref[...]
    if bn_relu:
        mu = jnp.mean(out, axis=0, keepdims=True)
        var = jnp.mean((out - mu) ** 2, axis=0, keepdims=True)
        out = g_ref[...] * (out - mu) * lax.rsqrt(var + 1e-5) + be_ref[...]
        out = jnp.maximum(out, 0.0)
    out_ref[...] = out


def _combine(s_parts, c_parts, x, root, bias, g, be, bn_relu):
    n, d = s_parts.shape[1], s_parts.shape[2]
    f = x.shape[1]
    full = lambda shape: pl.BlockSpec(shape, lambda: tuple(0 for _ in shape))
    return pl.pallas_call(
        functools.partial(_combine_body, bn_relu=bn_relu),
        in_specs=[full((n, d)), full((n, d)), full((n, d)), full((n, d)),
                  full((n, f)), full((f, d)), full((1, d)), full((1, d)),
                  full((1, d))],
        out_specs=full((n, d)),
        out_shape=jax.ShapeDtypeStruct((n, d), _F32),
    )(s_parts[0], s_parts[1], c_parts[0], c_parts[1], x, root, bias, g, be)


def _sc_gather(table, idx):
    n, d = table.shape
    e = idx.shape[0]
    nchunk = e // _CH
    iters = -(-nchunk // _NW)
    mesh = plsc.VectorSubcoreMesh(core_axis_name="c", subcore_axis_name="s")

    @functools.partial(
        pl.kernel,
        out_type=jax.ShapeDtypeStruct((e, d), _F32),
        mesh=mesh,
        scratch_types=[pltpu.VMEM((_CH,), jnp.int32),
                       pltpu.VMEM((_CH, d), _F32),
                       pltpu.SemaphoreType.DMA],
        compiler_params=pltpu.CompilerParams(use_tc_tiling_on_sc=False),
    )
    def gather_k(table_hbm, idx_hbm, out_hbm, idx_v, rows_v, sem):
        wid = lax.axis_index("s") * _NC + lax.axis_index("c")

        def chunk(j, carry):
            c = wid + j * _NW

            @pl.when(c < nchunk)
            def _():
                base = c * _CH
                pltpu.sync_copy(idx_hbm.at[pl.ds(base, _CH)], idx_v)
                pltpu.async_copy(table_hbm.at[idx_v], rows_v, sem).wait()
                pltpu.sync_copy(rows_v, out_hbm.at[pl.ds(base, _CH)])

            return carry

        lax.fori_loop(0, iters, chunk, 0)

    return gather_k(table, idx)


def _sc_scatter(msg, dst, n, with_counts):
    e, d = msg.shape
    nchunk = e // _CH
    iters = -(-nchunk // _NW)
    mesh = plsc.VectorSubcoreMesh(core_axis_name="c", subcore_axis_name="s")
    zeros = jnp.zeros((n, d), _F32)
    ones = jnp.ones((_CH, d), _F32)

    n_out = 2 if with_counts else 1
    out_type = [jax.ShapeDtypeStruct((_NC, n, d), _F32)] * n_out
    scratch = [pltpu.VMEM((_CH,), jnp.int32),
               pltpu.VMEM((_CH, d), _F32),
               pltpu.VMEM((_CH, d), _F32),
               pltpu.VMEM_SHARED((n, d), _F32),
               pltpu.VMEM_SHARED((n, d), _F32)]

    @functools.partial(
        pl.kernel, out_type=out_type, mesh=mesh, scratch_types=scratch,
        compiler_params=pltpu.CompilerParams(use_tc_tiling_on_sc=False))
    def scatter_k(msg_hbm, dst_hbm, zeros_hbm, ones_hbm, *rest):
        outs = rest[:n_out]
        idx_v, msg_v, ones_v, acc_s, acc_c = rest[n_out:]
        cid = lax.axis_index("c")
        sid = lax.axis_index("s")
        wid = sid * _NC + cid

        @pl.when(sid == 0)
        def _():
            pltpu.sync_copy(zeros_hbm, acc_s)
            if with_counts:
                pltpu.sync_copy(zeros_hbm, acc_c)

        if with_counts:
            pltpu.sync_copy(ones_hbm, ones_v)
        plsc.subcore_barrier()

        def chunk(j, carry):
            c = wid + j * _NW

            @pl.when(c < nchunk)
            def _():
                base = c * _CH
                pltpu.sync_copy(dst_hbm.at[pl.ds(base, _CH)], idx_v)
                pltpu.sync_copy(msg_hbm.at[pl.ds(base, _CH)], msg_v)
                pltpu.sync_copy(msg_v, acc_s.at[idx_v], add=True)
                if with_counts:
                    pltpu.sync_copy(ones_v, acc_c.at[idx_v], add=True)

            return carry

        lax.fori_loop(0, iters, chunk, 0)
        plsc.subcore_barrier()

        @pl.when(sid == 0)
        def _():
            pltpu.sync_copy(acc_s, outs[0].at[cid])
            if with_counts:
                pltpu.sync_copy(acc_c, outs[1].at[cid])

    return scatter_k(msg, dst, zeros, ones)


def _fold_bn(gram, colsum, e, w1, b1, g, be):
    m = colsum[0] / e
    cov = gram / e - jnp.outer(m, m)
    mu = m @ w1 + b1
    var = jnp.sum(w1 * (cov @ w1), axis=0)
    inv = g * lax.rsqrt(var + 1e-5)
    return w1 * inv[None, :], ((b1 - mu) * inv + be)[None, :]


def _bilinear_mats(w2, b2, in_c, out_c):
    dim = w2.shape[0]
    w2r = w2.reshape(dim, in_c, out_c).transpose(1, 0, 2).reshape(
        in_c, dim * out_c)
    expand = jnp.kron(jnp.eye(dim, dtype=_F32), jnp.ones((1, out_c), _F32))
    summ = jnp.kron(jnp.ones((dim, 1), _F32), jnp.eye(out_c, dtype=_F32))
    b2m = b2.reshape(in_c, out_c)
    return w2r, expand, summ, b2m


def kernel(x, edge_index, edge_attr, nn1_w1, nn1_b1, nn1_g, nn1_be, nn1_w2,
           nn1_b2, conv1_root, conv1_bias, nn2_w1, nn2_b1, nn2_g, nn2_be,
           nn2_w2, nn2_b2, conv2_root, conv2_bias, bn_g, bn_be):
    n, f_in = x.shape
    e, t = edge_attr.shape
    dim = nn1_w1.shape[1]
    src = edge_index[0]
    dst = edge_index[1]
    tile = 4000

    gram, colsum = _gram(edge_attr, tile)
    w1f_1, c1_1 = _fold_bn(gram, colsum, e, nn1_w1, nn1_b1, nn1_g, nn1_be)
    w1f_2, c1_2 = _fold_bn(gram, colsum, e, nn2_w1, nn2_b1, nn2_g, nn2_be)
    w2r_1, exp_1, sum_1, b2m_1 = _bilinear_mats(nn1_w2, nn1_b2, f_in, dim)
    w2r_2, exp_2, sum_2, b2m_2 = _bilinear_mats(nn2_w2, nn2_b2, dim, dim)

    xj = _sc_gather(x, src)
    msg1 = _edge_messages(edge_attr, xj, w1f_1, c1_1, w2r_1, exp_1, sum_1,
                          b2m_1, tile)
    s1, cnt = _sc_scatter(msg1, dst, n, with_counts=True)
    h = _combine(s1, cnt, x, conv1_root, conv1_bias[None, :], bn_g[None, :],
                 bn_be[None, :], bn_relu=True)

    hj = _sc_gather(h, src)
    msg2 = _edge_messages(edge_attr, hj, w1f_2, c1_2, w2r_2, exp_2, sum_2,
                          b2m_2, tile)
    s2, = _sc_scatter(msg2, dst, n, with_counts=False)
    out = _combine(s2, cnt, h, conv2_root, conv2_bias[None, :], bn_g[None, :],
                   bn_be[None, :], bn_relu=False)
    return out


# speedup vs baseline: 3.8588x; 1.1491x over previous
import functools

import jax
import jax.numpy as jnp
from jax import lax
from jax.experimental import pallas as pl
from jax.experimental.pallas import tpu as pltpu
from jax.experimental.pallas import tpu_sc as plsc

_NC = 2
_NS = 16
_NW = _NC * _NS
_CH = 128

_F32 = jnp.float32


def _gram_body(ea_ref, g_ref, s_ref):
    i = pl.program_id(0)

    @pl.when(i == 0)
    def _():
        g_ref[...] = jnp.zeros_like(g_ref)
        s_ref[...] = jnp.zeros_like(s_ref)

    ea = ea_ref[...]
    g_ref[...] += lax.dot_general(ea, ea, (((0,), (0,)), ((), ())),
                                  preferred_element_type=_F32)
    s_ref[...] += jnp.sum(ea, axis=0, keepdims=True)


def _gram(ea, tile):
    e, t = ea.shape
    return pl.pallas_call(
        _gram_body,
        grid=(e // tile,),
        in_specs=[pl.BlockSpec((tile, t), lambda i: (i, 0))],
        out_specs=[pl.BlockSpec((t, t), lambda i: (0, 0)),
                   pl.BlockSpec((1, t), lambda i: (0, 0))],
        out_shape=[jax.ShapeDtypeStruct((t, t), _F32),
                   jax.ShapeDtypeStruct((1, t), _F32)],
    )(ea)


def _edge_body(ea_ref, xj_ref, w1_ref, c1_ref, w2r_ref, exp_ref, sum_ref,
               b2m_ref, out_ref):
    ea = ea_ref[...]
    xj = xj_ref[...]
    h = jnp.maximum(
        jnp.dot(ea, w1_ref[...], preferred_element_type=_F32) + c1_ref[...],
        0.0)
    m2 = jnp.dot(xj, w2r_ref[...], preferred_element_type=_F32)
    hexp = jnp.dot(h, exp_ref[...], preferred_element_type=_F32)
    msg = jnp.dot(hexp * m2, sum_ref[...], preferred_element_type=_F32)
    out_ref[...] = msg + jnp.dot(xj, b2m_ref[...], preferred_element_type=_F32)


def _edge_messages(ea, xj, w1f, c1, w2r, expand, summ, b2m, tile):
    e, t = ea.shape
    f = xj.shape[1]
    dp = w2r.shape[1]
    d = summ.shape[1]
    small = lambda shape: pl.BlockSpec(shape, lambda i: (0, 0))
    return pl.pallas_call(
        _edge_body,
        grid=(e // tile,),
        in_specs=[pl.BlockSpec((tile, t), lambda i: (i, 0)),
                  pl.BlockSpec((tile, f), lambda i: (i, 0)),
                  small((t, d)), small((1, d)), small((f, dp)),
                  small((d, dp)), small((dp, d)), small((f, d))],
        out_specs=pl.BlockSpec((tile, d), lambda i: (i, 0)),
        out_shape=jax.ShapeDtypeStruct((e, d), _F32),
    )(ea, xj, w1f, c1, w2r, expand, summ, b2m)


def _combine_body(s0_ref, s1_ref, c0_ref, c1_ref, x_ref, root_ref, bias_ref,
                  g_ref, be_ref, out_ref, *, bn_relu):
    s = s0_ref[...] + s1_ref[...]
    cnt = (c0_ref[...] + c1_ref[...])[:, 0:1]
    agg = s / jnp.maximum(cnt, 1.0)
    out = agg + jnp.dot(x_ref[...], root_ref[...],
                        preferred_element_type=_F32) + bias_ref[...]
    if bn_relu:
        mu = jnp.mean(out, axis=0, keepdims=True)
        var = jnp.mean((out - mu) ** 2, axis=0, keepdims=True)
        out = g_ref[...] * (out - mu) * lax.rsqrt(var + 1e-5) + be_ref[...]
        out = jnp.maximum(out, 0.0)
    out_ref[...] = out


def _combine(s_parts, c_parts, x, root, bias, g, be, bn_relu):
    n, d = s_parts.shape[1], s_parts.shape[2]
    f = x.shape[1]
    full = lambda shape: pl.BlockSpec(shape, lambda: tuple(0 for _ in shape))
    return pl.pallas_call(
        functools.partial(_combine_body, bn_relu=bn_relu),
        in_specs=[full((n, d)), full((n, d)), full((n, d)), full((n, d)),
                  full((n, f)), full((f, d)), full((1, d)), full((1, d)),
                  full((1, d))],
        out_specs=full((n, d)),
        out_shape=jax.ShapeDtypeStruct((n, d), _F32),
    )(s_parts[0], s_parts[1], c_parts[0], c_parts[1], x, root, bias, g, be)


_K = 8


def _pad_idx(idx, nchunk_pad):
    return jnp.pad(idx, (0, nchunk_pad * _CH - idx.shape[0])).reshape(
        nchunk_pad, _CH)


def _sc_gather(table, idx):
    n, d = table.shape
    e = idx.shape[0]
    nchunk = e // _CH
    iters = -(-(-(-nchunk // _K)) // _NW)
    nchunk_pad = iters * _NW * _K
    idx2 = _pad_idx(idx, nchunk_pad)
    mesh = plsc.VectorSubcoreMesh(core_axis_name="c", subcore_axis_name="s")

    @functools.partial(
        pl.kernel,
        out_type=jax.ShapeDtypeStruct((e, d), _F32),
        mesh=mesh,
        scratch_types=[pltpu.VMEM((_K, _CH), jnp.int32),
                       pltpu.VMEM((_K * _CH, d), _F32),
                       pltpu.SemaphoreType.DMA,
                       pltpu.SemaphoreType.DMA],
        compiler_params=pltpu.CompilerParams(use_tc_tiling_on_sc=False),
    )
    def gather_k(table_hbm, idx_hbm, out_hbm, idx_v, rows_v, gsem, wsem):
        wid = lax.axis_index("s") * _NC + lax.axis_index("c")

        def group(i, carry):
            g = wid + i * _NW
            pltpu.sync_copy(idx_hbm.at[pl.ds(g * _K, _K)], idx_v)
            for j in range(_K):
                c = (g - _NW) * _K + j

                @pl.when(jnp.logical_and(i > 0, c < nchunk))
                def _(j=j, c=c):
                    pltpu.make_async_copy(
                        rows_v.at[pl.ds(j * _CH, _CH)],
                        out_hbm.at[pl.ds(c * _CH, _CH)], wsem).wait()

            for j in range(_K):
                c = g * _K + j

                @pl.when(c < nchunk)
                def _(j=j, c=c):
                    pltpu.async_copy(table_hbm.at[idx_v.at[j]],
                                     rows_v.at[pl.ds(j * _CH, _CH)], gsem)

            for j in range(_K):
                c = g * _K + j

                @pl.when(c < nchunk)
                def _(j=j, c=c):
                    pltpu.make_async_copy(
                        table_hbm.at[idx_v.at[j]],
                        rows_v.at[pl.ds(j * _CH, _CH)], gsem).wait()

            for j in range(_K):
                c = g * _K + j

                @pl.when(c < nchunk)
                def _(j=j, c=c):
                    pltpu.async_copy(rows_v.at[pl.ds(j * _CH, _CH)],
                                     out_hbm.at[pl.ds(c * _CH, _CH)], wsem)

            return carry

        lax.fori_loop(0, iters, group, 0)
        g = wid + (iters - 1) * _NW
        for j in range(_K):
            c = g * _K + j

            @pl.when(c < nchunk)
            def _(j=j, c=c):
                pltpu.make_async_copy(rows_v.at[pl.ds(j * _CH, _CH)],
                                      out_hbm.at[pl.ds(c * _CH, _CH)],
                                      wsem).wait()

    return gather_k(table, idx2)


def _sc_scatter(msg, dst, n, with_counts):
    e, d = msg.shape
    nchunk = e // _CH
    iters = -(-(-(-nchunk // _K)) // _NW)
    nchunk_pad = iters * _NW * _K
    dst2 = _pad_idx(dst, nchunk_pad)
    mesh = plsc.VectorSubcoreMesh(core_axis_name="c", subcore_axis_name="s")
    zeros = jnp.zeros((n, d), _F32)
    ones = jnp.ones((_CH, d), _F32)

    n_out = 2 if with_counts else 1
    out_type = [jax.ShapeDtypeStruct((_NC, n, d), _F32)] * n_out
    scratch = [pltpu.VMEM((_K, _CH), jnp.int32),
               pltpu.VMEM((_K * _CH, d), _F32),
               pltpu.VMEM((_CH, d), _F32),
               pltpu.VMEM_SHARED((n, d), _F32),
               pltpu.VMEM_SHARED((n, d), _F32),
               pltpu.SemaphoreType.DMA,
               pltpu.SemaphoreType.DMA]

    @functools.partial(
        pl.kernel, out_type=out_type, mesh=mesh, scratch_types=scratch,
        compiler_params=pltpu.CompilerParams(use_tc_tiling_on_sc=False))
    def scatter_k(msg_hbm, dst_hbm, zeros_hbm, ones_hbm, *rest):
        outs = rest[:n_out]
        idx_v, msg_v, ones_v, acc_s, acc_c, lsem, ssem = rest[n_out:]
        cid = lax.axis_index("c")
        sid = lax.axis_index("s")
        wid = sid * _NC + cid

        @pl.when(sid == 0)
        def _():
            pltpu.sync_copy(zeros_hbm, acc_s)
            if with_counts:
                pltpu.sync_copy(zeros_hbm, acc_c)

        if with_counts:
            pltpu.sync_copy(ones_hbm, ones_v)
        plsc.subcore_barrier()

        def group(i, carry):
            g = wid + i * _NW
            pltpu.sync_copy(dst_hbm.at[pl.ds(g * _K, _K)], idx_v)
            for j in range(_K):
                c = g * _K + j

                @pl.when(c < nchunk)
                def _(j=j, c=c):
                    pltpu.async_copy(msg_hbm.at[pl.ds(c * _CH, _CH)],
                                     msg_v.at[pl.ds(j * _CH, _CH)], lsem)

            for j in range(_K):
                c = g * _K + j

                @pl.when(c < nchunk)
                def _(j=j, c=c):
                    pltpu.make_async_copy(
                        msg_hbm.at[pl.ds(c * _CH, _CH)],
                        msg_v.at[pl.ds(j * _CH, _CH)], lsem).wait()

            for j in range(_K):
                c = g * _K + j

                @pl.when(c < nchunk)
                def _(j=j, c=c):
                    pltpu.async_copy(msg_v.at[pl.ds(j * _CH, _CH)],
                                     acc_s.at[idx_v.at[j]], ssem, add=True)
                    if with_counts:
                        pltpu.async_copy(ones_v, acc_c.at[idx_v.at[j]],
                                         ssem, add=True)

            for j in range(_K):
                c = g * _K + j

                @pl.when(c < nchunk)
                def _(j=j, c=c):
                    pltpu.make_async_copy(msg_v.at[pl.ds(j * _CH, _CH)],
                                          acc_s.at[idx_v.at[j]], ssem).wait()
                    if with_counts:
                        pltpu.make_async_copy(ones_v, acc_c.at[idx_v.at[j]],
                                              ssem).wait()

            return carry

        lax.fori_loop(0, iters, group, 0)
        plsc.subcore_barrier()

        @pl.when(sid == 0)
        def _():
            pltpu.sync_copy(acc_s, outs[0].at[cid])
            if with_counts:
                pltpu.sync_copy(acc_c, outs[1].at[cid])

    return scatter_k(msg, dst2, zeros, ones)


def _fold_bn(gram, colsum, e, w1, b1, g, be):
    m = colsum[0] / e
    cov = gram / e - jnp.outer(m, m)
    mu = m @ w1 + b1
    var = jnp.sum(w1 * (cov @ w1), axis=0)
    inv = g * lax.rsqrt(var + 1e-5)
    return w1 * inv[None, :], ((b1 - mu) * inv + be)[None, :]


def _bilinear_mats(w2, b2, in_c, out_c):
    dim = w2.shape[0]
    w2r = w2.reshape(dim, in_c, out_c).transpose(1, 0, 2).reshape(
        in_c, dim * out_c)
    expand = jnp.kron(jnp.eye(dim, dtype=_F32), jnp.ones((1, out_c), _F32))
    summ = jnp.kron(jnp.ones((dim, 1), _F32), jnp.eye(out_c, dtype=_F32))
    b2m = b2.reshape(in_c, out_c)
    return w2r, expand, summ, b2m


def kernel(x, edge_index, edge_attr, nn1_w1, nn1_b1, nn1_g, nn1_be, nn1_w2,
           nn1_b2, conv1_root, conv1_bias, nn2_w1, nn2_b1, nn2_g, nn2_be,
           nn2_w2, nn2_b2, conv2_root, conv2_bias, bn_g, bn_be):
    n, f_in = x.shape
    e, t = edge_attr.shape
    dim = nn1_w1.shape[1]
    src = edge_index[0]
    dst = edge_index[1]
    tile = 4000

    gram, colsum = _gram(edge_attr, tile)
    w1f_1, c1_1 = _fold_bn(gram, colsum, e, nn1_w1, nn1_b1, nn1_g, nn1_be)
    w1f_2, c1_2 = _fold_bn(gram, colsum, e, nn2_w1, nn2_b1, nn2_g, nn2_be)
    w2r_1, exp_1, sum_1, b2m_1 = _bilinear_mats(nn1_w2, nn1_b2, f_in, dim)
    w2r_2, exp_2, sum_2, b2m_2 = _bilinear_mats(nn2_w2, nn2_b2, dim, dim)

    xj = _sc_gather(x, src)
    msg1 = _edge_messages(edge_attr, xj, w1f_1, c1_1, w2r_1, exp_1, sum_1,
                          b2m_1, tile)
    s1, cnt = _sc_scatter(msg1, dst, n, with_counts=True)
    h = _combine(s1, cnt, x, conv1_root, conv1_bias[None, :], bn_g[None, :],
                 bn_be[None, :], bn_relu=True)

    hj = _sc_gather(h, src)
    msg2 = _edge_messages(edge_attr, hj, w1f_2, c1_2, w2r_2, exp_2, sum_2,
                          b2m_2, tile)
    s2, = _sc_scatter(msg2, dst, n, with_counts=False)
    out = _combine(s2, cnt, h, conv2_root, conv2_bias[None, :], bn_g[None, :],
                   bn_be[None, :], bn_relu=False)
    return out
